# baseline (device time: 710937 ns/iter reference)
import jax
import jax.numpy as jnp
from jax import lax
from jax.experimental import pallas as pl
from jax.experimental.pallas import tpu as pltpu

NC = 8


def kernel(x):
    x = x.astype(jnp.bfloat16)
    m, n = x.shape
    half = n // 2
    cm = m // NC

    def body(x_ref, out_ref, send_buf, in_sems, send_sems, recv_sems, local_sem):
        my_x = lax.axis_index("x")
        my_y = lax.axis_index("y")
        my_z = lax.axis_index("z")
        peer = (my_x, 1 - my_y, my_z)

        rdmas = []
        for c in range(NC):
            slot = c % 2
            if c >= 2:
                rdmas[c - 2].wait_send()
            load = pltpu.make_async_copy(
                x_ref.at[pl.ds(c * cm, cm), pl.ds((1 - my_y) * half, half)],
                send_buf.at[slot],
                in_sems.at[slot],
            )
            load.start()
            load.wait()
            rdma = pltpu.make_async_remote_copy(
                src_ref=send_buf.at[slot],
                dst_ref=out_ref.at[pl.ds(my_y * m + c * cm, cm), :],
                send_sem=send_sems.at[slot],
                recv_sem=recv_sems.at[c],
                device_id=peer,
                device_id_type=pl.DeviceIdType.MESH,
            )
            rdma.start()
            rdmas.append(rdma)

        local = pltpu.make_async_copy(
            x_ref.at[:, pl.ds(my_y * half, half)],
            out_ref.at[pl.ds(my_y * m, m), :],
            local_sem,
        )
        local.start()

        rdmas[NC - 2].wait_send()
        rdmas[NC - 1].wait_send()
        for c in range(NC):
            rdmas[c].wait_recv()
        local.wait()

    return pl.pallas_call(
        body,
        out_shape=jax.ShapeDtypeStruct((2 * m, half), jnp.bfloat16),
        in_specs=[pl.BlockSpec(memory_space=pl.ANY)],
        out_specs=pl.BlockSpec(memory_space=pl.ANY),
        scratch_shapes=[
            pltpu.VMEM((2, cm, half), jnp.bfloat16),
            pltpu.SemaphoreType.DMA((2,)),
            pltpu.SemaphoreType.DMA((2,)),
            pltpu.SemaphoreType.DMA((NC,)),
            pltpu.SemaphoreType.DMA,
        ],
    )(x)


# device time: 250308 ns/iter; 2.8402x vs baseline; 2.8402x over previous
import jax
import jax.numpy as jnp
from jax import lax
from jax.experimental import pallas as pl
from jax.experimental.pallas import tpu as pltpu


def kernel(x):
    x = x.astype(jnp.bfloat16)
    m, n = x.shape
    half = n // 2

    def body(x_ref, out_ref, send_buf, loc_buf,
             load_sems, store_sem, send_sem, recv_sem):
        my_x = lax.axis_index("x")
        my_y = lax.axis_index("y")
        my_z = lax.axis_index("z")
        peer = (my_x, 1 - my_y, my_z)

        load_send = pltpu.make_async_copy(
            x_ref.at[:, pl.ds((1 - my_y) * half, half)], send_buf,
            load_sems.at[0])
        load_loc = pltpu.make_async_copy(
            x_ref.at[:, pl.ds(my_y * half, half)], loc_buf,
            load_sems.at[1])
        load_send.start()
        load_loc.start()

        load_send.wait()
        rdma = pltpu.make_async_remote_copy(
            src_ref=send_buf,
            dst_ref=out_ref.at[pl.ds(my_y * m, m), :],
            send_sem=send_sem,
            recv_sem=recv_sem,
            device_id=peer,
            device_id_type=pl.DeviceIdType.MESH,
        )
        rdma.start()

        load_loc.wait()
        store_loc = pltpu.make_async_copy(
            loc_buf, out_ref.at[pl.ds(my_y * m, m), :], store_sem)
        store_loc.start()
        store_loc.wait()
        rdma.wait()

    return pl.pallas_call(
        body,
        out_shape=jax.ShapeDtypeStruct((2 * m, half), jnp.bfloat16),
        in_specs=[pl.BlockSpec(memory_space=pl.ANY)],
        out_specs=pl.BlockSpec(memory_space=pl.ANY),
        scratch_shapes=[
            pltpu.VMEM((m, half), jnp.bfloat16),
            pltpu.VMEM((m, half), jnp.bfloat16),
            pltpu.SemaphoreType.DMA((2,)),
            pltpu.SemaphoreType.DMA,
            pltpu.SemaphoreType.DMA,
            pltpu.SemaphoreType.DMA,
        ],
    )(x)


# device time: 212706 ns/iter; 3.3423x vs baseline; 1.1768x over previous
import jax
import jax.numpy as jnp
from jax import lax
from jax.experimental import pallas as pl
from jax.experimental.pallas import tpu as pltpu

NC = 8


def kernel(x):
    m, n = x.shape
    half = n // 2
    cm = m // NC

    def body(x_ref, out_ref, peer_f32, keep_f32, send_stage, loc_stage,
             peer_load_sems, keep_load_sems, store_sems, send_sems, recv_sems):
        my_x = lax.axis_index("x")
        my_y = lax.axis_index("y")
        my_z = lax.axis_index("z")
        peer = (my_x, 1 - my_y, my_z)
        peer_col = (1 - my_y) * half
        my_col = my_y * half

        def start_loads(c):
            slot = c % 2
            pltpu.make_async_copy(
                x_ref.at[pl.ds(c * cm, cm), pl.ds(peer_col, half)],
                peer_f32.at[slot], peer_load_sems.at[slot]).start()
            pltpu.make_async_copy(
                x_ref.at[pl.ds(c * cm, cm), pl.ds(my_col, half)],
                keep_f32.at[slot], keep_load_sems.at[slot]).start()

        start_loads(0)

        rdmas = []
        stores = []
        for c in range(NC):
            slot = c % 2
            pltpu.make_async_copy(
                x_ref.at[pl.ds(c * cm, cm), pl.ds(peer_col, half)],
                peer_f32.at[slot], peer_load_sems.at[slot]).wait()
            send_stage[pl.ds(c * cm, cm), :] = peer_f32[slot].astype(jnp.bfloat16)
            rdma = pltpu.make_async_remote_copy(
                src_ref=send_stage.at[pl.ds(c * cm, cm), :],
                dst_ref=out_ref.at[pl.ds(my_y * m + c * cm, cm), :],
                send_sem=send_sems.at[c],
                recv_sem=recv_sems.at[c],
                device_id=peer,
                device_id_type=pl.DeviceIdType.MESH,
            )
            rdma.start()
            rdmas.append(rdma)

            pltpu.make_async_copy(
                x_ref.at[pl.ds(c * cm, cm), pl.ds(my_col, half)],
                keep_f32.at[slot], keep_load_sems.at[slot]).wait()
            if c + 1 < NC:
                start_loads(c + 1)
            if c >= 2:
                stores[c - 2].wait()
            loc_stage[slot] = keep_f32[slot].astype(jnp.bfloat16)
            store = pltpu.make_async_copy(
                loc_stage.at[slot],
                out_ref.at[pl.ds(my_y * m + c * cm, cm), :],
                store_sems.at[slot])
            store.start()
            stores.append(store)

        for c in range(NC):
            rdmas[c].wait_send()
            rdmas[c].wait_recv()
        stores[NC - 2].wait()
        stores[NC - 1].wait()

    return pl.pallas_call(
        body,
        out_shape=jax.ShapeDtypeStruct((2 * m, half), jnp.bfloat16),
        in_specs=[pl.BlockSpec(memory_space=pl.ANY)],
        out_specs=pl.BlockSpec(memory_space=pl.ANY),
        scratch_shapes=[
            pltpu.VMEM((2, cm, half), jnp.float32),
            pltpu.VMEM((2, cm, half), jnp.float32),
            pltpu.VMEM((m, half), jnp.bfloat16),
            pltpu.VMEM((2, cm, half), jnp.bfloat16),
            pltpu.SemaphoreType.DMA((2,)),
            pltpu.SemaphoreType.DMA((2,)),
            pltpu.SemaphoreType.DMA((2,)),
            pltpu.SemaphoreType.DMA((NC,)),
            pltpu.SemaphoreType.DMA((NC,)),
        ],
        compiler_params=pltpu.CompilerParams(
            vmem_limit_bytes=64 * 1024 * 1024,
        ),
    )(x)


# device time: 208289 ns/iter; 3.4132x vs baseline; 1.0212x over previous
import jax
import jax.numpy as jnp
from jax import lax
from jax.experimental import pallas as pl
from jax.experimental.pallas import tpu as pltpu

NC = 8


def kernel(x):
    m, n = x.shape
    half = n // 2
    cm = m // NC

    def body(x_ref, out_ref, peer_f32, keep_f32, send_stage, loc_stage,
             peer_load_sems, keep_load_sems, store_sems, send_sems, recv_sems):
        my_x = lax.axis_index("x")
        my_y = lax.axis_index("y")
        my_z = lax.axis_index("z")
        peer = (my_x, 1 - my_y, my_z)
        peer_col = (1 - my_y) * half
        my_col = my_y * half

        barrier_sem = pltpu.get_barrier_semaphore()
        pl.semaphore_signal(
            barrier_sem, 1, device_id=peer,
            device_id_type=pl.DeviceIdType.MESH)
        pl.semaphore_wait(barrier_sem, 1)

        def start_loads(c):
            slot = c % 2
            pltpu.make_async_copy(
                x_ref.at[pl.ds(c * cm, cm), pl.ds(peer_col, half)],
                peer_f32.at[slot], peer_load_sems.at[slot]).start()
            pltpu.make_async_copy(
                x_ref.at[pl.ds(c * cm, cm), pl.ds(my_col, half)],
                keep_f32.at[slot], keep_load_sems.at[slot]).start()

        start_loads(0)

        rdmas = []
        stores = []
        for c in range(NC):
            slot = c % 2
            pltpu.make_async_copy(
                x_ref.at[pl.ds(c * cm, cm), pl.ds(peer_col, half)],
                peer_f32.at[slot], peer_load_sems.at[slot]).wait()
            send_stage[pl.ds(c * cm, cm), :] = peer_f32[slot].astype(jnp.bfloat16)
            rdma = pltpu.make_async_remote_copy(
                src_ref=send_stage.at[pl.ds(c * cm, cm), :],
                dst_ref=out_ref.at[pl.ds(my_y * m + c * cm, cm), :],
                send_sem=send_sems.at[c],
                recv_sem=recv_sems.at[c],
                device_id=peer,
                device_id_type=pl.DeviceIdType.MESH,
            )
            rdma.start()
            rdmas.append(rdma)

            pltpu.make_async_copy(
                x_ref.at[pl.ds(c * cm, cm), pl.ds(my_col, half)],
                keep_f32.at[slot], keep_load_sems.at[slot]).wait()
            if c + 1 < NC:
                start_loads(c + 1)
            if c >= 2:
                stores[c - 2].wait()
            loc_stage[slot] = keep_f32[slot].astype(jnp.bfloat16)
            store = pltpu.make_async_copy(
                loc_stage.at[slot],
                out_ref.at[pl.ds(my_y * m + c * cm, cm), :],
                store_sems.at[slot])
            store.start()
            stores.append(store)

        for c in range(NC):
            rdmas[c].wait_send()
            rdmas[c].wait_recv()
        stores[NC - 2].wait()
        stores[NC - 1].wait()

    return pl.pallas_call(
        body,
        out_shape=jax.ShapeDtypeStruct((2 * m, half), jnp.bfloat16),
        in_specs=[pl.BlockSpec(memory_space=pl.ANY)],
        out_specs=pl.BlockSpec(memory_space=pl.ANY),
        scratch_shapes=[
            pltpu.VMEM((2, cm, half), jnp.float32),
            pltpu.VMEM((2, cm, half), jnp.float32),
            pltpu.VMEM((m, half), jnp.bfloat16),
            pltpu.VMEM((2, cm, half), jnp.bfloat16),
            pltpu.SemaphoreType.DMA((2,)),
            pltpu.SemaphoreType.DMA((2,)),
            pltpu.SemaphoreType.DMA((2,)),
            pltpu.SemaphoreType.DMA((NC,)),
            pltpu.SemaphoreType.DMA((NC,)),
        ],
        compiler_params=pltpu.CompilerParams(
            vmem_limit_bytes=64 * 1024 * 1024,
            collective_id=0,
        ),
    )(x)


# device time: 207733 ns/iter; 3.4224x vs baseline; 1.0027x over previous
import jax
import jax.numpy as jnp
from jax import lax
from jax.experimental import pallas as pl
from jax.experimental.pallas import tpu as pltpu

NC = 16


def kernel(x):
    m, n = x.shape
    half = n // 2
    cm = m // NC

    def body(x_ref, out_ref, peer_f32, keep_f32, send_stage, loc_stage,
             peer_load_sems, keep_load_sems, store_sems, send_sems, recv_sems):
        my_x = lax.axis_index("x")
        my_y = lax.axis_index("y")
        my_z = lax.axis_index("z")
        peer = (my_x, 1 - my_y, my_z)
        peer_col = (1 - my_y) * half
        my_col = my_y * half

        barrier_sem = pltpu.get_barrier_semaphore()
        pl.semaphore_signal(
            barrier_sem, 1, device_id=peer,
            device_id_type=pl.DeviceIdType.MESH)
        pl.semaphore_wait(barrier_sem, 1)

        def start_loads(c):
            slot = c % 2
            pltpu.make_async_copy(
                x_ref.at[pl.ds(c * cm, cm), pl.ds(peer_col, half)],
                peer_f32.at[slot], peer_load_sems.at[slot]).start()
            pltpu.make_async_copy(
                x_ref.at[pl.ds(c * cm, cm), pl.ds(my_col, half)],
                keep_f32.at[slot], keep_load_sems.at[slot]).start()

        start_loads(0)

        rdmas = []
        stores = []
        for c in range(NC):
            slot = c % 2
            pltpu.make_async_copy(
                x_ref.at[pl.ds(c * cm, cm), pl.ds(peer_col, half)],
                peer_f32.at[slot], peer_load_sems.at[slot]).wait()
            send_stage[pl.ds(c * cm, cm), :] = peer_f32[slot].astype(jnp.bfloat16)
            rdma = pltpu.make_async_remote_copy(
                src_ref=send_stage.at[pl.ds(c * cm, cm), :],
                dst_ref=out_ref.at[pl.ds(my_y * m + c * cm, cm), :],
                send_sem=send_sems.at[c],
                recv_sem=recv_sems.at[c],
                device_id=peer,
                device_id_type=pl.DeviceIdType.MESH,
            )
            rdma.start()
            rdmas.append(rdma)

            pltpu.make_async_copy(
                x_ref.at[pl.ds(c * cm, cm), pl.ds(my_col, half)],
                keep_f32.at[slot], keep_load_sems.at[slot]).wait()
            if c + 1 < NC:
                start_loads(c + 1)
            if c >= 2:
                stores[c - 2].wait()
            loc_stage[slot] = keep_f32[slot].astype(jnp.bfloat16)
            store = pltpu.make_async_copy(
                loc_stage.at[slot],
                out_ref.at[pl.ds(my_y * m + c * cm, cm), :],
                store_sems.at[slot])
            store.start()
            stores.append(store)

        for c in range(NC):
            rdmas[c].wait_send()
            rdmas[c].wait_recv()
        stores[NC - 2].wait()
        stores[NC - 1].wait()

    return pl.pallas_call(
        body,
        out_shape=jax.ShapeDtypeStruct((2 * m, half), jnp.bfloat16),
        in_specs=[pl.BlockSpec(memory_space=pl.ANY)],
        out_specs=pl.BlockSpec(memory_space=pl.ANY),
        scratch_shapes=[
            pltpu.VMEM((2, cm, half), jnp.float32),
            pltpu.VMEM((2, cm, half), jnp.float32),
            pltpu.VMEM((m, half), jnp.bfloat16),
            pltpu.VMEM((2, cm, half), jnp.bfloat16),
            pltpu.SemaphoreType.DMA((2,)),
            pltpu.SemaphoreType.DMA((2,)),
            pltpu.SemaphoreType.DMA((2,)),
            pltpu.SemaphoreType.DMA((NC,)),
            pltpu.SemaphoreType.DMA((NC,)),
        ],
        compiler_params=pltpu.CompilerParams(
            vmem_limit_bytes=64 * 1024 * 1024,
            collective_id=0,
        ),
    )(x)
